# baseline (speedup 1.0000x reference)
import functools

import jax
import jax.numpy as jnp
from jax import lax
from jax.experimental import pallas as pl
from jax.experimental.pallas import tpu as pltpu
from jax.experimental.pallas import tpu_sc as plsc

B = 4096
S = 50
D = 128
C = 1000
CP = 1024

NC = 2
NS = 16
NW = NC * NS
BPW = B // NW
L = 16
DL = D // L


NB = 8
SACC = S


def _make_pool_body(start, bpw):
    def _pool_body(ids_hbm, table_hbm, out_hbm, idx_v, rows_v, pooled_v, sems):
        wid = lax.axis_index("s") * NC + lax.axis_index("c")
        base = wid * bpw
        pltpu.sync_copy(ids_hbm.at[pl.ds(start + base, bpw), :], idx_v)

        def fire(e, k):
            pltpu.make_async_copy(
                table_hbm.at[idx_v.at[jnp.minimum(e, bpw - 1), pl.ds(0, S)]],
                rows_v.at[k],
                sems.at[k],
            ).start()

        def accum(e, k):
            def row(r, accs):
                return tuple(
                    accs[d] + rows_v[k, r, pl.ds(d * L, L)] for d in range(DL)
                )

            accs = lax.fori_loop(
                0, S, row, tuple(jnp.zeros((L,), jnp.float32) for _ in range(DL))
            )
            for d in range(DL):
                pooled_v[e, pl.ds(d * L, L)] = accs[d]

        def wait(k):
            pltpu.make_async_copy(
                table_hbm.at[idx_v.at[0, pl.ds(0, S)]], rows_v.at[k], sems.at[k]
            ).wait()

        for k in range(NB - 1):
            fire(k, k)

        def group(g, carry):
            e0 = g * NB
            for k in range(NB):
                fire(e0 + k + NB - 1, (k + NB - 1) % NB)
                wait(k)
                accum(e0 + k, k)
            return carry

        lax.fori_loop(0, bpw // NB, group, 0)
        for k in range(NB - 1):
            wait(k)
        pltpu.sync_copy(pooled_v, out_hbm.at[pl.ds(base, bpw), :])

    return _pool_body


def _pool_part(ids, table, start, nb):
    bpw = nb // NW
    mesh = plsc.VectorSubcoreMesh(core_axis_name="c", subcore_axis_name="s")
    return pl.kernel(
        _make_pool_body(start, bpw),
        out_type=jax.ShapeDtypeStruct((nb, D), jnp.float32),
        mesh=mesh,
        scratch_types=[
            pltpu.VMEM((bpw, 128), jnp.int32),
            pltpu.VMEM((NB, S, D), jnp.float32),
            pltpu.VMEM((bpw, D), jnp.float32),
            pltpu.SemaphoreType.DMA((NB,)),
        ],
    )(ids, table)


BT = 2048


def _mm_body(wt_ref, p_ref, b_ref, o_ref):
    o_ref[...] = (
        lax.dot_general(
            wt_ref[...],
            p_ref[...] * (1.0 / S),
            (((1,), (1,)), ((), ())),
            preferred_element_type=jnp.float32,
        )
        + b_ref[...]
    )


def _matmul_t(w_t, pooled, b_col):
    return pl.pallas_call(
        _mm_body,
        grid=(B // BT,),
        in_specs=[
            pl.BlockSpec((C, D), lambda i: (0, 0)),
            pl.BlockSpec((BT, D), lambda i: (i, 0)),
            pl.BlockSpec((C, 1), lambda i: (0, 0)),
        ],
        out_specs=pl.BlockSpec((C, BT), lambda i: (0, i)),
        out_shape=jax.ShapeDtypeStruct((C, B), jnp.float32),
    )(w_t, pooled, b_col)


@jax.jit
def kernel(input_ids, table, W, b):
    ids = input_ids.astype(jnp.int32)
    ids_pad = jnp.pad(ids, ((0, 0), (0, 128 - S)))
    pooled = _pool_part(ids_pad, table, 0, B)
    logits_t = _matmul_t(W.T, pooled, b.reshape(C, 1))
    return logits_t.T


# speedup vs baseline: 1.0029x; 1.0029x over previous
import jax
import jax.numpy as jnp
from jax import lax
from jax.experimental import pallas as pl
from jax.experimental.pallas import tpu as pltpu
from jax.experimental.pallas import tpu_sc as plsc

B = 4096
S = 50
D = 128
C = 1000

NC = 2
NS = 16
NW = NC * NS
BPW = B // NW
L = 16
DL = D // L

NB = 8


def _pool_body(ids_hbm, table_hbm, out_hbm, idx_v, rows_v, pooled_v, sems):
    wid = lax.axis_index("s") * NC + lax.axis_index("c")
    base = wid * BPW
    pltpu.sync_copy(ids_hbm.at[pl.ds(base, BPW), :], idx_v)

    def fire(e, k):
        pltpu.make_async_copy(
            table_hbm.at[idx_v.at[jnp.minimum(e, BPW - 1), pl.ds(0, S)]],
            rows_v.at[k],
            sems.at[k],
        ).start()

    def accum(e, k):
        def row(r, accs):
            return tuple(
                accs[d] + rows_v[k, r, pl.ds(d * L, L)] for d in range(DL)
            )

        accs = lax.fori_loop(
            0, S, row, tuple(jnp.zeros((L,), jnp.float32) for _ in range(DL))
        )
        for d in range(DL):
            pooled_v[e, pl.ds(d * L, L)] = accs[d]

    def wait(k):
        pltpu.make_async_copy(
            table_hbm.at[idx_v.at[0, pl.ds(0, S)]], rows_v.at[k], sems.at[k]
        ).wait()

    for k in range(NB - 1):
        fire(k, k)

    def group(g, carry):
        e0 = g * NB
        for k in range(NB):
            fire(e0 + k + NB - 1, (k + NB - 1) % NB)
            wait(k)
            accum(e0 + k, k)
        return carry

    lax.fori_loop(0, BPW // NB, group, 0)
    for k in range(NB - 1):
        wait(k)
    pltpu.sync_copy(pooled_v, out_hbm.at[pl.ds(base, BPW), :])


def _pool(ids, table):
    mesh = plsc.VectorSubcoreMesh(core_axis_name="c", subcore_axis_name="s")
    return pl.kernel(
        _pool_body,
        out_type=jax.ShapeDtypeStruct((B, D), jnp.float32),
        mesh=mesh,
        scratch_types=[
            pltpu.VMEM((BPW, 128), jnp.int32),
            pltpu.VMEM((NB, S, D), jnp.float32),
            pltpu.VMEM((BPW, D), jnp.float32),
            pltpu.SemaphoreType.DMA((NB,)),
        ],
    )(ids, table)


BT = 2048


def _mm_body(wt_ref, p_ref, b_ref, o_ref):
    o_ref[...] = (
        lax.dot_general(
            wt_ref[...],
            p_ref[...] * (1.0 / S),
            (((1,), (1,)), ((), ())),
            preferred_element_type=jnp.float32,
        )
        + b_ref[...]
    )


def _matmul_t(w_t, pooled, b_col):
    return pl.pallas_call(
        _mm_body,
        grid=(B // BT,),
        in_specs=[
            pl.BlockSpec((C, D), lambda i: (0, 0)),
            pl.BlockSpec((BT, D), lambda i: (i, 0)),
            pl.BlockSpec((C, 1), lambda i: (0, 0)),
        ],
        out_specs=pl.BlockSpec((C, BT), lambda i: (0, i)),
        out_shape=jax.ShapeDtypeStruct((C, B), jnp.float32),
    )(w_t, pooled, b_col)


@jax.jit
def kernel(input_ids, table, W, b):
    ids = input_ids.astype(jnp.int32)
    ids_pad = jnp.pad(ids, ((0, 0), (0, 128 - S)))
    pooled = _pool(ids_pad, table)
    logits_t = _matmul_t(W.T, pooled, b.reshape(C, 1))
    return logits_t.T
